# baseline (speedup 1.0000x reference)
import functools

import jax
import jax.numpy as jnp
from jax import lax
from jax.experimental import pallas as pl
from jax.experimental.pallas import tpu as pltpu
from jax.experimental.pallas import tpu_sc as plsc

B = 1024
SS = 50
DX = 128
E = 178
P0 = DX
ROWS = 2 * SS - 1
NW = 32
BPW = B // NW
L = 16

OFF50 = (0, 16, 32, 34)
OFF49 = (0, 16, 32, 33)


def _sc_call(examples, labels, table):
    mesh = plsc.VectorSubcoreMesh(core_axis_name="c", subcore_axis_name="s")

    @functools.partial(
        pl.kernel,
        mesh=mesh,
        out_type=jax.ShapeDtypeStruct((B * ROWS, E), jnp.float32),
        compiler_params=pltpu.CompilerParams(use_tc_tiling_on_sc=False),
        scratch_types=[
            pltpu.VMEM((BPW, SS), jnp.int32),
            pltpu.VMEM((SS, E), jnp.float32),
            pltpu.VMEM((SS, E), jnp.float32),
            pltpu.VMEM((BPW, SS), jnp.int32),
            pltpu.VMEM((BPW, SS), jnp.int32),
        ],
    )
    def k(ex_hbm, lab_hbm, tab_hbm, out_hbm, lab_v, rows_v, even_v, ide_v, ido_v):
        wid = lax.axis_index("s") * 2 + lax.axis_index("c")
        b0 = wid * BPW
        lanes = lax.iota(jnp.int32, L)

        pltpu.sync_copy(lab_hbm.at[pl.ds(b0, BPW)], lab_v.at[...])

        def posrow(s, _):
            for off in (P0, P0 + 16, P0 + 32, P0 + 34):
                col = off + lanes
                even_v[s, pl.ds(off, L)] = jnp.where(
                    col == P0 + s, 1.0, 0.0).astype(jnp.float32)
            return 0
        lax.fori_loop(0, SS, posrow, 0)

        def idrow(j, _):
            base = (b0 + j) * ROWS
            for off in OFF50:
                ide_v[j, pl.ds(off, L)] = base + 2 * (off + lanes)
            for off in OFF50:
                ido_v[j, pl.ds(off, L)] = base + jnp.minimum(
                    1 + 2 * (off + lanes), ROWS - 1)
            return 0
        lax.fori_loop(0, BPW, idrow, 0)

        def body(j, _):
            pltpu.sync_copy(tab_hbm.at[lab_v.at[j]], rows_v.at[...])
            pltpu.sync_copy(ex_hbm.at[b0 + j],
                            even_v.at[pl.ds(0, SS), pl.ds(0, DX)])
            pltpu.sync_copy(rows_v.at[...], out_hbm.at[ido_v.at[j]])
            pltpu.sync_copy(even_v.at[...], out_hbm.at[ide_v.at[j]])
            return 0
        lax.fori_loop(0, BPW, body, 0)

    return k(examples, labels, table)


def kernel(examples, labels, label_embs):
    out = _sc_call(examples, labels.astype(jnp.int32), label_embs)
    return out.reshape(B, ROWS, E)


# speedup vs baseline: 1.7922x; 1.7922x over previous
import functools

import jax
import jax.numpy as jnp
from jax import lax
from jax.experimental import pallas as pl
from jax.experimental.pallas import tpu as pltpu
from jax.experimental.pallas import tpu_sc as plsc

B = 1024
SS = 50
DX = 128
E = 178
ROWS = 2 * SS - 1
NODD = SS - 1
NW = 32
BPW = B // NW
L = 16
OV = E - DX
HOT = DX - OV

OFF99 = (0, 16, 32, 48, 64, 80, 83)
OFF128 = (0, 16, 32, 48, 64, 80, 96, 112)
NBLK = 112


def _sc_call(exf, labp, tab_a, tab_b):
    mesh = plsc.VectorSubcoreMesh(core_axis_name="c", subcore_axis_name="s")

    @functools.partial(
        pl.kernel,
        mesh=mesh,
        out_type=(
            jax.ShapeDtypeStruct((B * ROWS, DX), jnp.float32),
            jax.ShapeDtypeStruct((B * ROWS, DX), jnp.float32),
        ),
        compiler_params=pltpu.CompilerParams(use_tc_tiling_on_sc=False),
        scratch_types=[
            pltpu.VMEM((BPW, DX), jnp.int32),
            pltpu.VMEM((NBLK, DX), jnp.float32),
            pltpu.VMEM((NBLK, DX), jnp.float32),
            pltpu.VMEM((BPW, ROWS), jnp.int32),
        ],
    )
    def k(exf_hbm, lab_hbm, ta_hbm, tb_hbm, outa_hbm, outb_hbm,
          lab_v, ca_v, cb_v, ids_v):
        wid = lax.axis_index("s") * 2 + lax.axis_index("c")
        b0 = wid * BPW
        lanes = lax.iota(jnp.int32, L)

        pltpu.sync_copy(lab_hbm.at[pl.ds(b0, BPW)], lab_v.at[...])

        def posrow(s, _):
            for off in OFF128:
                col = off + lanes
                cb_v[s, pl.ds(off, L)] = jnp.where(
                    col == HOT + s, 1.0, 0.0).astype(jnp.float32)
            return 0
        lax.fori_loop(0, SS, posrow, 0)

        def idrow(j, _):
            base = (b0 + j) * ROWS
            for off in OFF99:
                col = off + lanes
                ids_v[j, pl.ds(off, L)] = base + jnp.where(
                    col < SS, 2 * col, 2 * (col - SS) + 1)
            return 0
        lax.fori_loop(0, BPW, idrow, 0)

        def body(j, _):
            for dst, tab in ((ca_v, ta_hbm), (cb_v, tb_hbm)):
                pltpu.sync_copy(tab.at[lab_v.at[j, pl.ds(0, 48)]],
                                dst.at[pl.ds(SS, 48)])
                pltpu.sync_copy(tab.at[lab_v.at[j, pl.ds(48, 8)]],
                                dst.at[pl.ds(SS + 48, 8)])
            pltpu.sync_copy(exf_hbm.at[pl.ds((b0 + j) * SS, SS)],
                            ca_v.at[pl.ds(0, SS)])
            pltpu.sync_copy(ca_v.at[pl.ds(0, ROWS)], outa_hbm.at[ids_v.at[j]])
            pltpu.sync_copy(cb_v.at[pl.ds(0, ROWS)], outb_hbm.at[ids_v.at[j]])
            return 0
        lax.fori_loop(0, BPW, body, 0)

    return k(exf, labp, tab_a, tab_b)


def kernel(examples, labels, label_embs):
    exf = examples.reshape(B * SS, DX)
    lab32 = labels.astype(jnp.int32)
    labp = jnp.concatenate([lab32, lab32, lab32[:, : DX - 2 * SS]], axis=1)
    tab_a = label_embs[:, :DX]
    tab_b = label_embs[:, OV:E]
    outa, outb = _sc_call(exf, labp, tab_a, tab_b)
    hh = jnp.concatenate([outa, outb[:, HOT:]], axis=1)
    return hh.reshape(B, ROWS, E)


# speedup vs baseline: 2.0649x; 1.1522x over previous
import functools

import jax
import jax.numpy as jnp
from jax import lax
from jax.experimental import pallas as pl
from jax.experimental.pallas import tpu as pltpu
from jax.experimental.pallas import tpu_sc as plsc

B = 1024
SS = 50
DX = 128
E = 178
ROWS = 2 * SS - 1
NODD = SS - 1
NW = 32
BPW = B // NW
L = 16
OV = E - DX
HOT = DX - OV

OFF99 = (0, 16, 32, 48, 64, 80, 83)
OFF128 = (0, 16, 32, 48, 64, 80, 96, 112)
NBLK = 112


def _sc_call(exf, labp, tab_a, tab_b):
    mesh = plsc.VectorSubcoreMesh(core_axis_name="c", subcore_axis_name="s")

    @functools.partial(
        pl.kernel,
        mesh=mesh,
        out_type=(
            jax.ShapeDtypeStruct((B * ROWS, DX), jnp.float32),
            jax.ShapeDtypeStruct((B * ROWS, DX), jnp.float32),
        ),
        compiler_params=pltpu.CompilerParams(use_tc_tiling_on_sc=False),
        scratch_types=[
            pltpu.VMEM((BPW, DX), jnp.int32),
            pltpu.VMEM((NBLK, DX), jnp.float32),
            pltpu.VMEM((NBLK, DX), jnp.float32),
            pltpu.VMEM((BPW, ROWS), jnp.int32),
        ],
    )
    def k(exf_hbm, lab_hbm, ta_hbm, tb_hbm, outa_hbm, outb_hbm,
          lab_v, ca_v, cb_v, ids_v):
        wid = lax.axis_index("s") * 2 + lax.axis_index("c")
        b0 = wid * BPW
        lanes = lax.iota(jnp.int32, L)

        pltpu.sync_copy(lab_hbm.at[pl.ds(b0, BPW)], lab_v.at[...])

        def posrow(s, _):
            for off in OFF128:
                col = off + lanes
                cb_v[s, pl.ds(off, L)] = jnp.where(
                    col == HOT + s, 1.0, 0.0).astype(jnp.float32)
            return 0
        lax.fori_loop(0, SS, posrow, 0)

        def idrow(j, _):
            base = (b0 + j) * ROWS
            for off in OFF99:
                col = off + lanes
                ids_v[j, pl.ds(off, L)] = base + jnp.where(
                    col < SS, 2 * col, 2 * (col - SS) + 1)
            return 0
        lax.fori_loop(0, BPW, idrow, 0)

        def body(j, _):
            def inner(isem, osem):
                ing = [
                    pltpu.async_copy(ta_hbm.at[lab_v.at[j, pl.ds(0, 56)]],
                                     ca_v.at[pl.ds(SS, 56)], isem),
                    pltpu.async_copy(tb_hbm.at[lab_v.at[j, pl.ds(0, 56)]],
                                     cb_v.at[pl.ds(SS, 56)], isem),
                    pltpu.async_copy(exf_hbm.at[pl.ds((b0 + j) * SS, SS)],
                                     ca_v.at[pl.ds(0, SS)], isem),
                ]
                for c in ing:
                    c.wait()
                outg = [
                    pltpu.async_copy(ca_v.at[pl.ds(0, ROWS)],
                                     outa_hbm.at[ids_v.at[j]], osem),
                    pltpu.async_copy(cb_v.at[pl.ds(0, ROWS)],
                                     outb_hbm.at[ids_v.at[j]], osem),
                ]
                for c in outg:
                    c.wait()
            pl.run_scoped(inner, pltpu.SemaphoreType.DMA(()),
                          pltpu.SemaphoreType.DMA(()))
            return 0
        lax.fori_loop(0, BPW, body, 0)

    return k(exf, labp, tab_a, tab_b)


def kernel(examples, labels, label_embs):
    exf = examples.reshape(B * SS, DX)
    lab32 = labels.astype(jnp.int32)
    labp = jnp.concatenate([lab32, lab32, lab32[:, : DX - 2 * SS]], axis=1)
    tab_a = label_embs[:, :DX]
    tab_b = label_embs[:, OV:E]
    outa, outb = _sc_call(exf, labp, tab_a, tab_b)
    hh = jnp.concatenate([outa, outb[:, HOT:]], axis=1)
    return hh.reshape(B, ROWS, E)


# speedup vs baseline: 2.1329x; 1.0329x over previous
import functools

import jax
import jax.numpy as jnp
from jax import lax
from jax.experimental import pallas as pl
from jax.experimental.pallas import tpu as pltpu
from jax.experimental.pallas import tpu_sc as plsc

B = 1024
SS = 50
DX = 128
E = 178
ROWS = 2 * SS - 1
NODD = SS - 1
NW = 32
BPW = B // NW
L = 16
OV = E - DX
HOT = DX - OV

OFF99 = (0, 16, 32, 48, 64, 80, 83)
OFF128 = (0, 16, 32, 48, 64, 80, 96, 112)
NBLK = 112


def _sc_call(exf, labp, tab_a, tab_b):
    mesh = plsc.VectorSubcoreMesh(core_axis_name="c", subcore_axis_name="s")

    @functools.partial(
        pl.kernel,
        mesh=mesh,
        out_type=(
            jax.ShapeDtypeStruct((B * ROWS, DX), jnp.float32),
            jax.ShapeDtypeStruct((B * ROWS, DX), jnp.float32),
        ),
        compiler_params=pltpu.CompilerParams(use_tc_tiling_on_sc=False),
        scratch_types=[
            pltpu.VMEM((BPW, DX), jnp.int32),
            pltpu.VMEM((NBLK, DX), jnp.float32),
            pltpu.VMEM((NBLK, DX), jnp.float32),
            pltpu.VMEM((NBLK, DX), jnp.float32),
            pltpu.VMEM((NBLK, DX), jnp.float32),
            pltpu.VMEM((BPW, ROWS), jnp.int32),
        ],
    )
    def k(exf_hbm, lab_hbm, ta_hbm, tb_hbm, outa_hbm, outb_hbm,
          lab_v, ca0_v, ca1_v, cb0_v, cb1_v, ids_v):
        wid = lax.axis_index("s") * 2 + lax.axis_index("c")
        b0 = wid * BPW
        lanes = lax.iota(jnp.int32, L)

        pltpu.sync_copy(lab_hbm.at[pl.ds(b0, BPW)], lab_v.at[...])

        def posrow(s, _):
            for off in OFF128:
                col = off + lanes
                v = jnp.where(col == HOT + s, 1.0, 0.0).astype(jnp.float32)
                cb0_v[s, pl.ds(off, L)] = v
                cb1_v[s, pl.ds(off, L)] = v
            return 0
        lax.fori_loop(0, SS, posrow, 0)

        def idrow(j, _):
            base = (b0 + j) * ROWS
            for off in OFF99:
                col = off + lanes
                ids_v[j, pl.ds(off, L)] = base + jnp.where(
                    col < SS, 2 * col, 2 * (col - SS) + 1)
            return 0
        lax.fori_loop(0, BPW, idrow, 0)

        def fire_in(j, ca, cb, isem):
            pltpu.async_copy(ta_hbm.at[lab_v.at[j, pl.ds(0, 56)]],
                             ca.at[pl.ds(SS, 56)], isem)
            pltpu.async_copy(tb_hbm.at[lab_v.at[j, pl.ds(0, 56)]],
                             cb.at[pl.ds(SS, 56)], isem)
            pltpu.async_copy(exf_hbm.at[pl.ds((b0 + j) * SS, SS)],
                             ca.at[pl.ds(0, SS)], isem)

        def drain_in(j, ca, cb, isem):
            pltpu.make_async_copy(ta_hbm.at[lab_v.at[j, pl.ds(0, 56)]],
                                  ca.at[pl.ds(SS, 56)], isem).wait()
            pltpu.make_async_copy(tb_hbm.at[lab_v.at[j, pl.ds(0, 56)]],
                                  cb.at[pl.ds(SS, 56)], isem).wait()
            pltpu.make_async_copy(exf_hbm.at[pl.ds((b0 + j) * SS, SS)],
                                  ca.at[pl.ds(0, SS)], isem).wait()

        def fire_out(j, ca, cb, osem):
            pltpu.async_copy(ca.at[pl.ds(0, ROWS)],
                             outa_hbm.at[ids_v.at[j]], osem)
            pltpu.async_copy(cb.at[pl.ds(0, ROWS)],
                             outb_hbm.at[ids_v.at[j]], osem)

        def drain_out(j, ca, cb, osem):
            pltpu.make_async_copy(ca.at[pl.ds(0, ROWS)],
                                  outa_hbm.at[ids_v.at[j]], osem).wait()
            pltpu.make_async_copy(cb.at[pl.ds(0, ROWS)],
                                  outb_hbm.at[ids_v.at[j]], osem).wait()

        def pipeline(i0, i1, o0, o1):
            fire_in(0, ca0_v, cb0_v, i0)
            drain_in(0, ca0_v, cb0_v, i0)
            fire_out(0, ca0_v, cb0_v, o0)
            fire_in(1, ca1_v, cb1_v, i1)
            drain_in(1, ca1_v, cb1_v, i1)
            fire_out(1, ca1_v, cb1_v, o1)
            drain_out(0, ca0_v, cb0_v, o0)
            fire_in(2, ca0_v, cb0_v, i0)

            def body(i, _):
                j0 = 2 * i
                j1 = 2 * i + 1
                drain_in(j0, ca0_v, cb0_v, i0)
                fire_out(j0, ca0_v, cb0_v, o0)
                drain_out(j1 - 2, ca1_v, cb1_v, o1)
                fire_in(j0 + 1, ca1_v, cb1_v, i1)
                drain_in(j1, ca1_v, cb1_v, i1)
                fire_out(j1, ca1_v, cb1_v, o1)
                drain_out(j1 - 1, ca0_v, cb0_v, o0)
                fire_in(j1 + 1, ca0_v, cb0_v, i0)
                return 0
            lax.fori_loop(1, BPW // 2 - 1, body, 0)

            j = BPW - 2
            drain_in(j, ca0_v, cb0_v, i0)
            fire_out(j, ca0_v, cb0_v, o0)
            drain_out(j - 1, ca1_v, cb1_v, o1)
            fire_in(j + 1, ca1_v, cb1_v, i1)
            drain_in(j + 1, ca1_v, cb1_v, i1)
            fire_out(j + 1, ca1_v, cb1_v, o1)
            drain_out(j, ca0_v, cb0_v, o0)
            drain_out(j + 1, ca1_v, cb1_v, o1)

        pl.run_scoped(pipeline,
                      pltpu.SemaphoreType.DMA(()), pltpu.SemaphoreType.DMA(()),
                      pltpu.SemaphoreType.DMA(()), pltpu.SemaphoreType.DMA(()))

    return k(exf, labp, tab_a, tab_b)


def kernel(examples, labels, label_embs):
    exf = examples.reshape(B * SS, DX)
    lab32 = labels.astype(jnp.int32)
    labp = jnp.concatenate([lab32, lab32, lab32[:, : DX - 2 * SS]], axis=1)
    tab_a = label_embs[:, :DX]
    tab_b = label_embs[:, OV:E]
    outa, outb = _sc_call(exf, labp, tab_a, tab_b)
    hh = jnp.concatenate([outa, outb[:, HOT:]], axis=1)
    return hh.reshape(B, ROWS, E)


# speedup vs baseline: 2.1367x; 1.0018x over previous
import functools

import jax
import jax.numpy as jnp
from jax import lax
from jax.experimental import pallas as pl
from jax.experimental.pallas import tpu as pltpu
from jax.experimental.pallas import tpu_sc as plsc

B = 1024
SS = 50
DX = 128
E = 178
ROWS = 2 * SS - 1
NODD = SS - 1
NW = 32
BPW = B // NW
L = 16
OV = E - DX
HOT = DX - OV

OFF99 = (0, 16, 32, 48, 64, 80, 83)
OFF128 = (0, 16, 32, 48, 64, 80, 96, 112)
NBLK = 112


def _sc_call(exf, labp, tab_a, tab_b):
    mesh = plsc.VectorSubcoreMesh(core_axis_name="c", subcore_axis_name="s")

    @functools.partial(
        pl.kernel,
        mesh=mesh,
        out_type=(
            jax.ShapeDtypeStruct((B * ROWS, DX), jnp.float32),
            jax.ShapeDtypeStruct((B * ROWS, DX), jnp.float32),
        ),
        compiler_params=pltpu.CompilerParams(use_tc_tiling_on_sc=False),
        scratch_types=[
            pltpu.VMEM((BPW, DX), jnp.int32),
            pltpu.VMEM((NBLK, DX), jnp.float32),
            pltpu.VMEM((NBLK, DX), jnp.float32),
            pltpu.VMEM((NBLK, DX), jnp.float32),
            pltpu.VMEM((NBLK, DX), jnp.float32),
            pltpu.VMEM((BPW, ROWS), jnp.int32),
        ],
    )
    def k(exf_hbm, lab_hbm, ta_hbm, tb_hbm, outa_hbm, outb_hbm,
          lab_v, ca0_v, ca1_v, cb0_v, cb1_v, ids_v):
        wid = lax.axis_index("s") * 2 + lax.axis_index("c")
        b0 = wid * BPW
        lanes = lax.iota(jnp.int32, L)

        pltpu.sync_copy(lab_hbm.at[pl.ds(b0, BPW)], lab_v.at[...])

        def posrow(s, _):
            for off in OFF128:
                col = off + lanes
                v = jnp.where(col == HOT + s, 1.0, 0.0).astype(jnp.float32)
                cb0_v[s, pl.ds(off, L)] = v
                cb1_v[s, pl.ds(off, L)] = v
            return 0
        lax.fori_loop(0, SS, posrow, 0)

        def idrow(j, _):
            base = (b0 + j) * ROWS
            for off in OFF99:
                col = off + lanes
                ids_v[j, pl.ds(off, L)] = base + jnp.where(
                    col < SS, 2 * col, 2 * (col - SS) + 1)
            return 0
        lax.fori_loop(0, BPW, idrow, 0)

        def fire_in(j, ca, cb, isem):
            pltpu.async_copy(ta_hbm.at[lab_v.at[j, pl.ds(0, 56)]],
                             ca.at[pl.ds(SS, 56)], isem)
            pltpu.async_copy(tb_hbm.at[lab_v.at[j, pl.ds(0, 56)]],
                             cb.at[pl.ds(SS, 56)], isem)
            pltpu.async_copy(exf_hbm.at[pl.ds((b0 + j) * SS, SS)],
                             ca.at[pl.ds(0, SS)], isem)

        def drain_in(j, ca, cb, isem):
            pltpu.make_async_copy(ta_hbm.at[lab_v.at[j, pl.ds(0, 56)]],
                                  ca.at[pl.ds(SS, 56)], isem).wait()
            pltpu.make_async_copy(tb_hbm.at[lab_v.at[j, pl.ds(0, 56)]],
                                  cb.at[pl.ds(SS, 56)], isem).wait()
            pltpu.make_async_copy(exf_hbm.at[pl.ds((b0 + j) * SS, SS)],
                                  ca.at[pl.ds(0, SS)], isem).wait()

        def fire_out(j, ca, cb, osem):
            pltpu.async_copy(ca.at[pl.ds(0, ROWS)],
                             outa_hbm.at[ids_v.at[j]], osem)
            pltpu.async_copy(cb.at[pl.ds(0, ROWS)],
                             outb_hbm.at[ids_v.at[j]], osem)

        def drain_out(j, ca, cb, osem):
            pltpu.make_async_copy(ca.at[pl.ds(0, ROWS)],
                                  outa_hbm.at[ids_v.at[j]], osem).wait()
            pltpu.make_async_copy(cb.at[pl.ds(0, ROWS)],
                                  outb_hbm.at[ids_v.at[j]], osem).wait()

        def pipeline(i0, i1, o0, o1):
            fire_in(0, ca0_v, cb0_v, i0)
            drain_in(0, ca0_v, cb0_v, i0)
            fire_out(0, ca0_v, cb0_v, o0)
            fire_in(1, ca1_v, cb1_v, i1)
            drain_in(1, ca1_v, cb1_v, i1)
            fire_out(1, ca1_v, cb1_v, o1)
            drain_out(0, ca0_v, cb0_v, o0)
            fire_in(2, ca0_v, cb0_v, i0)

            def body(i, _):
                j0 = 2 * i
                j1 = 2 * i + 1
                drain_in(j0, ca0_v, cb0_v, i0)
                fire_out(j0, ca0_v, cb0_v, o0)
                drain_out(j1 - 2, ca1_v, cb1_v, o1)
                fire_in(j0 + 1, ca1_v, cb1_v, i1)
                drain_in(j1, ca1_v, cb1_v, i1)
                fire_out(j1, ca1_v, cb1_v, o1)
                drain_out(j1 - 1, ca0_v, cb0_v, o0)
                fire_in(j1 + 1, ca0_v, cb0_v, i0)
                return 0
            lax.fori_loop(1, BPW // 2 - 1, body, 0)

            j = BPW - 2
            drain_in(j, ca0_v, cb0_v, i0)
            fire_out(j, ca0_v, cb0_v, o0)
            drain_out(j - 1, ca1_v, cb1_v, o1)
            fire_in(j + 1, ca1_v, cb1_v, i1)
            drain_in(j + 1, ca1_v, cb1_v, i1)
            fire_out(j + 1, ca1_v, cb1_v, o1)
            drain_out(j, ca0_v, cb0_v, o0)
            drain_out(j + 1, ca1_v, cb1_v, o1)

        pl.run_scoped(pipeline,
                      pltpu.SemaphoreType.DMA(()), pltpu.SemaphoreType.DMA(()),
                      pltpu.SemaphoreType.DMA(()), pltpu.SemaphoreType.DMA(()))

    return k(exf, labp, tab_a, tab_b)


def kernel(examples, labels, label_embs):
    lab32 = labels.astype(jnp.int32)
    one = (1 + lab32[0, 0] * 0).astype(jnp.float32)
    exf = examples.reshape(B * SS, DX) * one
    labp = jnp.concatenate([lab32, lab32, lab32[:, : DX - 2 * SS]], axis=1)
    tab_a = label_embs[:, :DX]
    tab_b = label_embs[:, OV:E]
    outa, outb = _sc_call(exf, labp, tab_a, tab_b)
    hh = jnp.concatenate([outa, outb[:, HOT:]], axis=1)
    return hh.reshape(B, ROWS, E)
